# baseline (speedup 1.0000x reference)
import functools

import jax
import jax.numpy as jnp
from jax import lax
from jax.experimental import pallas as pl
from jax.experimental.pallas import tpu as pltpu
from jax.experimental.pallas import tpu_sc as plsc

NN = 10000
EE = 320000
RR = 4
BB = 4
HH = 128

NC = 2
NS = 16
NWORK = NC * NS

CH = 96
NCH = 108
EPW = NCH * CH
E_PAD = EPW * NWORK

NN_PAD = 10240
ROWS_PER_SUB = NN_PAD // NS
NROW_STAGE = 64
BN = 1000


def _apply_bases(x, bases_ref, comp_ref, y_ref):
    zs = [jnp.dot(x, bases_ref[b], preferred_element_type=jnp.float32)
          for b in range(BB)]
    for r in range(RR):
        acc = zs[0] * comp_ref[r, 0]
        for b in range(1, BB):
            acc = acc + zs[b] * comp_ref[r, b]
        y_ref[r] = acc


def _transform1_body(x_ref, bases_ref, comp_ref, y_ref):
    _apply_bases(x_ref[...], bases_ref, comp_ref, y_ref)


def _transform2_body(hp_ref, bias_ref, bases_ref, comp_ref, y_ref):
    x = jnp.maximum(hp_ref[0] + hp_ref[1] + bias_ref[...], 0.0)
    _apply_bases(x, bases_ref, comp_ref, y_ref)


def _final_body(hp_ref, bias_ref, o_ref):
    o_ref[...] = hp_ref[0] + hp_ref[1] + bias_ref[...]


def _tc_transform1(x, bases, comp):
    return pl.pallas_call(
        _transform1_body,
        grid=(NN // BN,),
        in_specs=[
            pl.BlockSpec((BN, HH), lambda i: (i, 0)),
            pl.BlockSpec((BB, HH, HH), lambda i: (0, 0, 0)),
            pl.BlockSpec(memory_space=pltpu.SMEM),
        ],
        out_specs=pl.BlockSpec((RR, BN, HH), lambda i: (0, i, 0)),
        out_shape=jax.ShapeDtypeStruct((RR, NN, HH), jnp.float32),
    )(x, bases, comp)


def _tc_transform2(hp, bias, bases, comp):
    return pl.pallas_call(
        _transform2_body,
        grid=(NN // BN,),
        in_specs=[
            pl.BlockSpec((2, BN, HH), lambda i: (0, i, 0)),
            pl.BlockSpec((1, HH), lambda i: (0, 0)),
            pl.BlockSpec((BB, HH, HH), lambda i: (0, 0, 0)),
            pl.BlockSpec(memory_space=pltpu.SMEM),
        ],
        out_specs=pl.BlockSpec((RR, BN, HH), lambda i: (0, i, 0)),
        out_shape=jax.ShapeDtypeStruct((RR, NN, HH), jnp.float32),
    )(hp, bias, bases, comp)


def _tc_final(hp, bias):
    return pl.pallas_call(
        _final_body,
        grid=(NN // BN,),
        in_specs=[
            pl.BlockSpec((2, BN, HH), lambda i: (0, i, 0)),
            pl.BlockSpec((1, HH), lambda i: (0, 0)),
        ],
        out_specs=pl.BlockSpec((BN, HH), lambda i: (i, 0)),
        out_shape=jax.ShapeDtypeStruct((NN, HH), jnp.float32),
    )(hp, bias)


_SC_MESH = plsc.VectorSubcoreMesh(core_axis_name="c", subcore_axis_name="s")

NBUF = 3
ISLOT = 6
NGRP = NCH // ISLOT

_SPLAT_DNUMS = lax.GatherDimensionNumbers(
    offset_dims=(), collapsed_slice_dims=(0,), start_index_map=(0,))


@functools.partial(
    pl.kernel,
    out_type=jax.ShapeDtypeStruct((NC * NN_PAD, HH), jnp.float32),
    mesh=_SC_MESH,
    scratch_types=[
        pltpu.VMEM_SHARED((NN_PAD, HH), jnp.float32),
        pltpu.VMEM((ISLOT, CH), jnp.int32),
        pltpu.VMEM((ISLOT, CH), jnp.int32),
        pltpu.VMEM((ISLOT, CH), jnp.float32),
        pltpu.VMEM((NBUF, CH, HH), jnp.float32),
        pltpu.VMEM((NROW_STAGE, HH), jnp.float32),
        pltpu.SemaphoreType.DMA((NBUF,)),
        pltpu.SemaphoreType.DMA((NBUF,)),
        pltpu.SemaphoreType.DMA((ISLOT,)),
        pltpu.SemaphoreType.DMA((ISLOT,)),
        pltpu.SemaphoreType.DMA((ISLOT,)),
    ],
)
def _sc_edge_aggregate(y_hbm, gidx_hbm, dst_hbm, norm_hbm, out_hbm,
                       h_sc, gidx_r, dst_r, norm_r, rows, stage_v,
                       gsem, ssem, isem_g, isem_d, isem_n):
    cid = lax.axis_index("c")
    sid = lax.axis_index("s")
    wid = sid * NC + cid
    row0 = sid * ROWS_PER_SUB

    zero16 = jnp.zeros((16,), jnp.float32)

    def _zrow(i, carry):
        for j in range(HH // 16):
            stage_v[i, pl.ds(j * 16, 16)] = zero16
        return carry

    lax.fori_loop(0, NROW_STAGE, _zrow, 0)
    for q in range(ROWS_PER_SUB // NROW_STAGE):
        pltpu.sync_copy(stage_v, h_sc.at[pl.ds(row0 + q * NROW_STAGE, NROW_STAGE)])
    plsc.subcore_barrier()

    def _idx_copies(k, slot):
        return (pltpu.make_async_copy(gidx_hbm.at[wid, k], gidx_r.at[slot],
                                      isem_g.at[slot]),
                pltpu.make_async_copy(dst_hbm.at[wid, k], dst_r.at[slot],
                                      isem_d.at[slot]),
                pltpu.make_async_copy(norm_hbm.at[wid, k], norm_r.at[slot],
                                      isem_n.at[slot]))

    def _gather_copy(slot, b):
        return pltpu.make_async_copy(y_hbm.at[gidx_r.at[slot]], rows.at[b],
                                     gsem.at[b])

    def _scatter_copy(slot, b):
        return pltpu.make_async_copy(rows.at[b], h_sc.at[dst_r.at[slot]],
                                     ssem.at[b])

    def _scale(slot, b):
        def _blk(blk, c2):
            nv = norm_r[slot, pl.ds(blk * 16, 16)]
            for i2 in range(16):
                sp = lax.gather(nv, jnp.full((16, 1), i2, jnp.int32),
                                _SPLAT_DNUMS, (1,),
                                mode=lax.GatherScatterMode.PROMISE_IN_BOUNDS)
                i = blk * 16 + i2
                for j in range(HH // 16):
                    sel = pl.ds(j * 16, 16)
                    rows[b, i, sel] = rows[b, i, sel] * sp
            return c2

        lax.fori_loop(0, CH // 16, _blk, 0)

    for t in range(NBUF):
        for c in _idx_copies(jnp.int32(t), t):
            c.start()
    for c in _idx_copies(jnp.int32(0), 0):
        c.wait()
    _gather_copy(0, 0).start()

    def _group(g, carry):
        for b in range(ISLOT):
            k = g * ISLOT + b
            rb = b % NBUF
            _gather_copy(b, rb).wait()

            @pl.when(k >= 2)
            def _retire():
                _scatter_copy((b + 4) % ISLOT, (b + 1) % NBUF).wait()

            @pl.when(k + 1 < NCH)
            def _refill():
                for c in _idx_copies(k + 1, (b + 1) % ISLOT):
                    c.wait()
                _gather_copy((b + 1) % ISLOT, (b + 1) % NBUF).start()

            @pl.when(k + 3 < NCH)
            def _prefetch():
                for c in _idx_copies(k + 3, (b + 3) % ISLOT):
                    c.start()

            _scale(b, rb)
            _scatter_copy(b, rb).start(add=True)
        return carry

    lax.fori_loop(0, NGRP, _group, 0)
    _scatter_copy((NCH - 2) % ISLOT, (NCH - 2) % NBUF).wait()
    _scatter_copy((NCH - 1) % ISLOT, (NCH - 1) % NBUF).wait()
    plsc.subcore_barrier()

    for q in range(ROWS_PER_SUB // NROW_STAGE):
        r = row0 + q * NROW_STAGE
        pltpu.sync_copy(h_sc.at[pl.ds(r, NROW_STAGE)], stage_v)
        pltpu.sync_copy(stage_v, out_hbm.at[pl.ds(cid * NN_PAD + r, NROW_STAGE)])


def kernel(edge_index, etype, norm, emb_weight,
           bases1, comp1, bias1, bases2, comp2, bias2):
    src = edge_index[0].astype(jnp.int32)
    dst = edge_index[1].astype(jnp.int32)
    gidx = etype.astype(jnp.int32) * NN + src
    order = jnp.argsort(gidx)
    gidx = gidx[order]
    dst = dst[order]
    norm = norm.reshape(EE)[order]

    pad = E_PAD - EE
    zi = jnp.zeros((pad,), jnp.int32)
    gidx_p = jnp.concatenate([gidx, zi]).reshape(NWORK, NCH, CH)
    dst_p = jnp.concatenate([dst, zi]).reshape(NWORK, NCH, CH)
    norm_p = jnp.concatenate(
        [norm.reshape(EE), jnp.zeros((pad,), jnp.float32)]).reshape(NWORK, NCH, CH)

    y1 = _tc_transform1(emb_weight, bases1, comp1).reshape(RR * NN, HH)
    hp1 = _sc_edge_aggregate(y1, gidx_p, dst_p, norm_p)
    y2 = _tc_transform2(hp1.reshape(NC, NN_PAD, HH), bias1.reshape(1, HH),
                        bases2, comp2).reshape(RR * NN, HH)
    hp2 = _sc_edge_aggregate(y2, gidx_p, dst_p, norm_p)
    return _tc_final(hp2.reshape(NC, NN_PAD, HH), bias2.reshape(1, HH))


# speedup vs baseline: 4.9987x; 4.9987x over previous
import functools

import jax
import jax.numpy as jnp
from jax import lax
from jax.experimental import pallas as pl
from jax.experimental.pallas import tpu as pltpu
from jax.experimental.pallas import tpu_sc as plsc

NN = 10000
EE = 320000
RR = 4
BB = 4
HH = 128

NC = 2
NS = 16
NWORK = NC * NS

CH = 96
NCH = 108
EPW = NCH * CH
E_PAD = EPW * NWORK

NN_PAD = 10240
ROWS_PER_SUB = NN_PAD // NS
NROW_STAGE = 64
BN = 1000


def _apply_bases(x, bases_ref, comp_ref, y_ref):
    zs = [jnp.dot(x, bases_ref[b], preferred_element_type=jnp.float32)
          for b in range(BB)]
    for r in range(RR):
        acc = zs[0] * comp_ref[r, 0]
        for b in range(1, BB):
            acc = acc + zs[b] * comp_ref[r, b]
        y_ref[r] = acc


def _transform1_body(x_ref, bases_ref, comp_ref, y_ref):
    _apply_bases(x_ref[...], bases_ref, comp_ref, y_ref)


def _transform2_body(hp_ref, bias_ref, bases_ref, comp_ref, y_ref):
    x = jnp.maximum(hp_ref[0] + hp_ref[1] + bias_ref[...], 0.0)
    _apply_bases(x, bases_ref, comp_ref, y_ref)


def _final_body(hp_ref, bias_ref, o_ref):
    o_ref[...] = hp_ref[0] + hp_ref[1] + bias_ref[...]


def _tc_transform1(x, bases, comp):
    return pl.pallas_call(
        _transform1_body,
        grid=(NN // BN,),
        in_specs=[
            pl.BlockSpec((BN, HH), lambda i: (i, 0)),
            pl.BlockSpec((BB, HH, HH), lambda i: (0, 0, 0)),
            pl.BlockSpec(memory_space=pltpu.SMEM),
        ],
        out_specs=pl.BlockSpec((RR, BN, HH), lambda i: (0, i, 0)),
        out_shape=jax.ShapeDtypeStruct((RR, NN, HH), jnp.float32),
    )(x, bases, comp)


def _tc_transform2(hp, bias, bases, comp):
    return pl.pallas_call(
        _transform2_body,
        grid=(NN // BN,),
        in_specs=[
            pl.BlockSpec((2, BN, HH), lambda i: (0, i, 0)),
            pl.BlockSpec((1, HH), lambda i: (0, 0)),
            pl.BlockSpec((BB, HH, HH), lambda i: (0, 0, 0)),
            pl.BlockSpec(memory_space=pltpu.SMEM),
        ],
        out_specs=pl.BlockSpec((RR, BN, HH), lambda i: (0, i, 0)),
        out_shape=jax.ShapeDtypeStruct((RR, NN, HH), jnp.float32),
    )(hp, bias, bases, comp)


def _tc_final(hp, bias):
    return pl.pallas_call(
        _final_body,
        grid=(NN // BN,),
        in_specs=[
            pl.BlockSpec((2, BN, HH), lambda i: (0, i, 0)),
            pl.BlockSpec((1, HH), lambda i: (0, 0)),
        ],
        out_specs=pl.BlockSpec((BN, HH), lambda i: (i, 0)),
        out_shape=jax.ShapeDtypeStruct((NN, HH), jnp.float32),
    )(hp, bias)


_SC_MESH = plsc.VectorSubcoreMesh(core_axis_name="c", subcore_axis_name="s")

NBUF = 3
ISLOT = 6
NGRP = NCH // ISLOT

_SPLAT_DNUMS = lax.GatherDimensionNumbers(
    offset_dims=(), collapsed_slice_dims=(0,), start_index_map=(0,))


@functools.partial(
    pl.kernel,
    out_type=jax.ShapeDtypeStruct((NC * NN_PAD, HH), jnp.float32),
    mesh=_SC_MESH,
    scratch_types=[
        pltpu.VMEM_SHARED((NN_PAD, HH), jnp.float32),
        pltpu.VMEM((ISLOT, CH), jnp.int32),
        pltpu.VMEM((ISLOT, CH), jnp.int32),
        pltpu.VMEM((ISLOT, CH), jnp.float32),
        pltpu.VMEM((NBUF, CH, HH), jnp.float32),
        pltpu.VMEM((NROW_STAGE, HH), jnp.float32),
        pltpu.SemaphoreType.DMA((NBUF,)),
        pltpu.SemaphoreType.DMA((NBUF,)),
        pltpu.SemaphoreType.DMA((ISLOT,)),
        pltpu.SemaphoreType.DMA((ISLOT,)),
        pltpu.SemaphoreType.DMA((ISLOT,)),
    ],
)
def _sc_edge_aggregate(y_hbm, gidx_hbm, dst_hbm, norm_hbm, out_hbm,
                       h_sc, gidx_r, dst_r, norm_r, rows, stage_v,
                       gsem, ssem, isem_g, isem_d, isem_n):
    cid = lax.axis_index("c")
    sid = lax.axis_index("s")
    wid = sid * NC + cid
    row0 = sid * ROWS_PER_SUB

    zero16 = jnp.zeros((16,), jnp.float32)

    def _zrow(i, carry):
        for j in range(HH // 16):
            stage_v[i, pl.ds(j * 16, 16)] = zero16
        return carry

    lax.fori_loop(0, NROW_STAGE, _zrow, 0)
    for q in range(ROWS_PER_SUB // NROW_STAGE):
        pltpu.sync_copy(stage_v, h_sc.at[pl.ds(row0 + q * NROW_STAGE, NROW_STAGE)])
    plsc.subcore_barrier()

    def _idx_copies(k, slot):
        return (pltpu.make_async_copy(gidx_hbm.at[wid, k], gidx_r.at[slot],
                                      isem_g.at[slot]),
                pltpu.make_async_copy(dst_hbm.at[wid, k], dst_r.at[slot],
                                      isem_d.at[slot]),
                pltpu.make_async_copy(norm_hbm.at[wid, k], norm_r.at[slot],
                                      isem_n.at[slot]))

    def _gather_copy(slot, b):
        return pltpu.make_async_copy(y_hbm.at[gidx_r.at[slot]], rows.at[b],
                                     gsem.at[b])

    def _scatter_copy(slot, b):
        return pltpu.make_async_copy(rows.at[b], h_sc.at[dst_r.at[slot]],
                                     ssem.at[b])

    def _scale(slot, b):
        def _blk(blk, c2):
            nv = norm_r[slot, pl.ds(blk * 16, 16)]
            for i2 in range(16):
                sp = lax.gather(nv, jnp.full((16, 1), i2, jnp.int32),
                                _SPLAT_DNUMS, (1,),
                                mode=lax.GatherScatterMode.PROMISE_IN_BOUNDS)
                i = blk * 16 + i2
                for j in range(HH // 16):
                    sel = pl.ds(j * 16, 16)
                    rows[b, i, sel] = rows[b, i, sel] * sp
            return c2

        lax.fori_loop(0, CH // 16, _blk, 0)

    for t in range(NBUF):
        for c in _idx_copies(jnp.int32(t), t):
            c.start()
    for c in _idx_copies(jnp.int32(0), 0):
        c.wait()
    _gather_copy(0, 0).start()

    def _group(g, carry):
        for b in range(ISLOT):
            k = g * ISLOT + b
            rb = b % NBUF
            _gather_copy(b, rb).wait()

            @pl.when(k >= 2)
            def _retire():
                _scatter_copy((b + 4) % ISLOT, (b + 1) % NBUF).wait()

            @pl.when(k + 1 < NCH)
            def _refill():
                for c in _idx_copies(k + 1, (b + 1) % ISLOT):
                    c.wait()
                _gather_copy((b + 1) % ISLOT, (b + 1) % NBUF).start()

            @pl.when(k + 3 < NCH)
            def _prefetch():
                for c in _idx_copies(k + 3, (b + 3) % ISLOT):
                    c.start()

            _scale(b, rb)
            _scatter_copy(b, rb).start(add=True)
        return carry

    lax.fori_loop(0, NGRP, _group, 0)
    _scatter_copy((NCH - 2) % ISLOT, (NCH - 2) % NBUF).wait()
    _scatter_copy((NCH - 1) % ISLOT, (NCH - 1) % NBUF).wait()
    plsc.subcore_barrier()

    for q in range(ROWS_PER_SUB // NROW_STAGE):
        r = row0 + q * NROW_STAGE
        pltpu.sync_copy(h_sc.at[pl.ds(r, NROW_STAGE)], stage_v)
        pltpu.sync_copy(stage_v, out_hbm.at[pl.ds(cid * NN_PAD + r, NROW_STAGE)])


def kernel(edge_index, etype, norm, emb_weight,
           bases1, comp1, bias1, bases2, comp2, bias2):
    src = edge_index[0].astype(jnp.int32)
    dst = edge_index[1].astype(jnp.int32)
    gidx = etype.astype(jnp.int32) * NN + src

    pad = E_PAD - EE
    zi = jnp.zeros((pad,), jnp.int32)
    pad_gidx = jnp.arange(pad, dtype=jnp.int32) % (RR * NN)
    gidx_p = jnp.concatenate([gidx, pad_gidx]).reshape(NWORK, NCH, CH)
    dst_p = jnp.concatenate([dst, zi]).reshape(NWORK, NCH, CH)
    norm_p = jnp.concatenate(
        [norm.reshape(EE), jnp.zeros((pad,), jnp.float32)]).reshape(NWORK, NCH, CH)

    y1 = _tc_transform1(emb_weight, bases1, comp1).reshape(RR * NN, HH)
    hp1 = _sc_edge_aggregate(y1, gidx_p, dst_p, norm_p)
    y2 = _tc_transform2(hp1.reshape(NC, NN_PAD, HH), bias1.reshape(1, HH),
                        bases2, comp2).reshape(RR * NN, HH)
    hp2 = _sc_edge_aggregate(y2, gidx_p, dst_p, norm_p)
    return _tc_final(hp2.reshape(NC, NN_PAD, HH), bias2.reshape(1, HH))
